# baseline (device time: 214514 ns/iter reference)
import jax
import jax.numpy as jnp
from jax import lax
from jax.experimental import pallas as pl
from jax.experimental.pallas import tpu as pltpu

N_DEV = 8


def kernel(x, W1, W2):
    m_per, d_model = x.shape
    f_per = W1.shape[1]

    def body(x_ref, W1_ref, W2_ref, out_ref, xg_ref, rs_recv_ref, acc_ref,
             ag_send_sems, ag_recv_sems, rs_send_sems, rs_recv_sems):
        my = lax.axis_index("i")
        left = lax.rem(my - 1 + N_DEV, N_DEV)
        right = lax.rem(my + 1, N_DEV)

        barrier_sem = pltpu.get_barrier_semaphore()
        for nbr in (left, right):
            pl.semaphore_signal(
                barrier_sem, inc=1,
                device_id=(nbr,), device_id_type=pl.DeviceIdType.MESH,
            )
        pl.semaphore_wait(barrier_sem, 2)

        def contribution(xblk):
            h = jnp.dot(xblk, W1_ref[...], preferred_element_type=jnp.float32)
            h = h * jax.nn.sigmoid(h)
            return jnp.dot(h, W2_ref[...], preferred_element_type=jnp.float32)

        xg_ref[my] = x_ref[...]
        for h in range(N_DEV - 1):
            send_blk = (my - h) % N_DEV
            rdma = pltpu.make_async_remote_copy(
                src_ref=xg_ref.at[send_blk],
                dst_ref=xg_ref.at[send_blk],
                send_sem=ag_send_sems.at[h],
                recv_sem=ag_recv_sems.at[h],
                device_id=(right,),
                device_id_type=pl.DeviceIdType.MESH,
            )
            rdma.start()
            rdma.wait()

        for s in range(N_DEV - 1):
            c = (my - 1 - s) % N_DEV
            part = contribution(xg_ref[c])
            if s == 0:
                acc_ref[...] = part
            else:
                acc_ref[...] = part + rs_recv_ref[s - 1]
            rdma = pltpu.make_async_remote_copy(
                src_ref=acc_ref,
                dst_ref=rs_recv_ref.at[s],
                send_sem=rs_send_sems.at[s],
                recv_sem=rs_recv_sems.at[s],
                device_id=(right,),
                device_id_type=pl.DeviceIdType.MESH,
            )
            rdma.start()
            rdma.wait()

        out_ref[...] = contribution(x_ref[...]) + rs_recv_ref[N_DEV - 2]

    return pl.pallas_call(
        body,
        out_shape=jax.ShapeDtypeStruct((m_per, d_model), jnp.float32),
        in_specs=[
            pl.BlockSpec(memory_space=pltpu.VMEM),
            pl.BlockSpec(memory_space=pltpu.VMEM),
            pl.BlockSpec(memory_space=pltpu.VMEM),
        ],
        out_specs=pl.BlockSpec(memory_space=pltpu.VMEM),
        scratch_shapes=[
            pltpu.VMEM((N_DEV, m_per, d_model), jnp.float32),
            pltpu.VMEM((N_DEV - 1, m_per, d_model), jnp.float32),
            pltpu.VMEM((m_per, d_model), jnp.float32),
            pltpu.SemaphoreType.DMA((N_DEV - 1,)),
            pltpu.SemaphoreType.DMA((N_DEV - 1,)),
            pltpu.SemaphoreType.DMA((N_DEV - 1,)),
            pltpu.SemaphoreType.DMA((N_DEV - 1,)),
        ],
        compiler_params=pltpu.CompilerParams(collective_id=0),
    )(x, W1, W2)


# device time: 99007 ns/iter; 2.1667x vs baseline; 2.1667x over previous
import jax
import jax.numpy as jnp
from jax import lax
from jax.experimental import pallas as pl
from jax.experimental.pallas import tpu as pltpu

N_DEV = 8


def kernel(x, W1, W2):
    m_per, d_model = x.shape
    half = m_per // 2

    def body(x_ref, W1_ref, W2_ref, out_ref, xgR, xgL, rsR, rsL, accR, accL,
             agR_ssem, agR_rsem, agL_ssem, agL_rsem,
             rsR_ssem, rsR_rsem, rsL_ssem, rsL_rsem):
        my = lax.axis_index("i")
        left = lax.rem(my - 1 + N_DEV, N_DEV)
        right = lax.rem(my + 1, N_DEV)

        barrier_sem = pltpu.get_barrier_semaphore()
        for nbr in (left, right):
            pl.semaphore_signal(
                barrier_sem, inc=1,
                device_id=(nbr,), device_id_type=pl.DeviceIdType.MESH,
            )
        pl.semaphore_wait(barrier_sem, 2)

        def contribution(xblk):
            h = jnp.dot(xblk, W1_ref[...], preferred_element_type=jnp.float32)
            h = h * jax.nn.sigmoid(h)
            return jnp.dot(h, W2_ref[...], preferred_element_type=jnp.float32)

        def ag_rdma(h):
            blkR = (my - h) % N_DEV
            aR = pltpu.make_async_remote_copy(
                src_ref=xgR.at[blkR], dst_ref=xgR.at[blkR],
                send_sem=agR_ssem.at[h], recv_sem=agR_rsem.at[h],
                device_id=(right,), device_id_type=pl.DeviceIdType.MESH,
            )
            blkL = (my + h) % N_DEV
            aL = pltpu.make_async_remote_copy(
                src_ref=xgL.at[blkL], dst_ref=xgL.at[blkL],
                send_sem=agL_ssem.at[h], recv_sem=agL_rsem.at[h],
                device_id=(left,), device_id_type=pl.DeviceIdType.MESH,
            )
            return aR, aL

        def rs_rdma(s):
            rR = pltpu.make_async_remote_copy(
                src_ref=accR.at[s % 2], dst_ref=rsR.at[s],
                send_sem=rsR_ssem.at[s], recv_sem=rsR_rsem.at[s],
                device_id=(right,), device_id_type=pl.DeviceIdType.MESH,
            )
            rL = pltpu.make_async_remote_copy(
                src_ref=accL.at[s % 2], dst_ref=rsL.at[s],
                send_sem=rsL_ssem.at[s], recv_sem=rsL_rsem.at[s],
                device_id=(left,), device_id_type=pl.DeviceIdType.MESH,
            )
            return rR, rL

        xgR[my] = x_ref[:half, :]
        xgL[my] = x_ref[half:, :]
        aR0, aL0 = ag_rdma(0)
        aR0.start()
        aL0.start()

        for s in range(N_DEV - 1):
            aRs, aLs = ag_rdma(s)
            aRs.wait_recv()
            aLs.wait_recv()
            if s < N_DEV - 2:
                aRn, aLn = ag_rdma(s + 1)
                aRn.start()
                aLn.start()

            partR = contribution(xgR[(my - 1 - s) % N_DEV])
            if s > 0:
                rR_prev, rL_prev = rs_rdma(s - 1)
                rR_prev.wait_recv()
                partR = partR + rsR[s - 1]
            if s > 1:
                rR_old, rL_old = rs_rdma(s - 2)
                rR_old.wait_send()
            accR[s % 2] = partR
            rRs, rLs = rs_rdma(s)
            rRs.start()

            partL = contribution(xgL[(my + 1 + s) % N_DEV])
            if s > 0:
                rL_prev.wait_recv()
                partL = partL + rsL[s - 1]
            if s > 1:
                rL_old.wait_send()
            accL[s % 2] = partL
            rLs.start()

        pT = contribution(x_ref[:half, :])
        pB = contribution(x_ref[half:, :])
        rR_last, rL_last = rs_rdma(N_DEV - 2)
        rR_last.wait_recv()
        out_ref[:half, :] = pT + rsR[N_DEV - 2]
        rL_last.wait_recv()
        out_ref[half:, :] = pB + rsL[N_DEV - 2]

        for h in range(N_DEV - 1):
            aR, aL = ag_rdma(h)
            aR.wait_send()
            aL.wait_send()
        for s in (N_DEV - 3, N_DEV - 2):
            rR, rL = rs_rdma(s)
            rR.wait_send()
            rL.wait_send()

    return pl.pallas_call(
        body,
        out_shape=jax.ShapeDtypeStruct((m_per, d_model), jnp.float32),
        in_specs=[
            pl.BlockSpec(memory_space=pltpu.VMEM),
            pl.BlockSpec(memory_space=pltpu.VMEM),
            pl.BlockSpec(memory_space=pltpu.VMEM),
        ],
        out_specs=pl.BlockSpec(memory_space=pltpu.VMEM),
        scratch_shapes=[
            pltpu.VMEM((N_DEV, half, d_model), jnp.float32),
            pltpu.VMEM((N_DEV, half, d_model), jnp.float32),
            pltpu.VMEM((N_DEV - 1, half, d_model), jnp.float32),
            pltpu.VMEM((N_DEV - 1, half, d_model), jnp.float32),
            pltpu.VMEM((2, half, d_model), jnp.float32),
            pltpu.VMEM((2, half, d_model), jnp.float32),
            pltpu.SemaphoreType.DMA((N_DEV - 1,)),
            pltpu.SemaphoreType.DMA((N_DEV - 1,)),
            pltpu.SemaphoreType.DMA((N_DEV - 1,)),
            pltpu.SemaphoreType.DMA((N_DEV - 1,)),
            pltpu.SemaphoreType.DMA((N_DEV - 1,)),
            pltpu.SemaphoreType.DMA((N_DEV - 1,)),
            pltpu.SemaphoreType.DMA((N_DEV - 1,)),
            pltpu.SemaphoreType.DMA((N_DEV - 1,)),
        ],
        compiler_params=pltpu.CompilerParams(collective_id=0),
    )(x, W1, W2)


# device time: 97179 ns/iter; 2.2074x vs baseline; 1.0188x over previous
import jax
import jax.numpy as jnp
from jax import lax
from jax.experimental import pallas as pl
from jax.experimental.pallas import tpu as pltpu

N_DEV = 8


def kernel(x, W1, W2):
    m_per, d_model = x.shape
    half = m_per // 2

    def body(x_ref, W1_ref, W2_ref, out_ref, xgR, xgL, rsR, rsL, accR, accL,
             agR_ssem, agR_rsem, agL_ssem, agL_rsem,
             rsR_ssem, rsR_rsem, rsL_ssem, rsL_rsem):
        my = lax.axis_index("i")
        left = lax.rem(my - 1 + N_DEV, N_DEV)
        right = lax.rem(my + 1, N_DEV)

        barrier_sem = pltpu.get_barrier_semaphore()
        for nbr in (left, right):
            pl.semaphore_signal(
                barrier_sem, inc=1,
                device_id=(nbr,), device_id_type=pl.DeviceIdType.MESH,
            )
        pl.semaphore_wait(barrier_sem, 2)

        def contribution(xblk):
            h = jnp.dot(xblk, W1_ref[...], preferred_element_type=jnp.float32)
            h = h * jax.nn.sigmoid(h)
            return jnp.dot(h, W2_ref[...], preferred_element_type=jnp.float32)

        def ag_rdma(h):
            blkR = (my - h) % N_DEV
            aR = pltpu.make_async_remote_copy(
                src_ref=xgR.at[blkR], dst_ref=xgR.at[blkR],
                send_sem=agR_ssem.at[h], recv_sem=agR_rsem.at[h],
                device_id=(right,), device_id_type=pl.DeviceIdType.MESH,
            )
            blkL = (my + h) % N_DEV
            aL = pltpu.make_async_remote_copy(
                src_ref=xgL.at[blkL], dst_ref=xgL.at[blkL],
                send_sem=agL_ssem.at[h], recv_sem=agL_rsem.at[h],
                device_id=(left,), device_id_type=pl.DeviceIdType.MESH,
            )
            return aR, aL

        def rs_rdma(s):
            rR = pltpu.make_async_remote_copy(
                src_ref=accR.at[s % 2], dst_ref=rsR.at[s],
                send_sem=rsR_ssem.at[s], recv_sem=rsR_rsem.at[s],
                device_id=(right,), device_id_type=pl.DeviceIdType.MESH,
            )
            rL = pltpu.make_async_remote_copy(
                src_ref=accL.at[s % 2], dst_ref=rsL.at[s],
                send_sem=rsL_ssem.at[s], recv_sem=rsL_rsem.at[s],
                device_id=(left,), device_id_type=pl.DeviceIdType.MESH,
            )
            return rR, rL

        xgR[my] = x_ref[:half, :]
        xgL[my] = x_ref[half:, :]
        aR0, aL0 = ag_rdma(0)
        aR0.start()
        aL0.start()

        for s in range(N_DEV - 1):
            aRs, aLs = ag_rdma(s)
            aRs.wait_recv()
            aLs.wait_recv()
            if s < N_DEV - 2:
                aRn, aLn = ag_rdma(s + 1)
                aRn.start()
                aLn.start()

            partR = contribution(xgR[(my - 1 - s) % N_DEV])
            partL = contribution(xgL[(my + 1 + s) % N_DEV])
            if s == 0:
                out_ref[:half, :] = contribution(x_ref[:half, :])
                out_ref[half:, :] = contribution(x_ref[half:, :])

            if s > 1:
                rR_old, rL_old = rs_rdma(s - 2)
                rR_old.wait_send()
                rL_old.wait_send()
            if s > 0:
                rR_prev, rL_prev = rs_rdma(s - 1)
                rR_prev.wait_recv()
                partR = partR + rsR[s - 1]
            accR[s % 2] = partR
            rRs, rLs = rs_rdma(s)
            rRs.start()
            if s > 0:
                rL_prev.wait_recv()
                partL = partL + rsL[s - 1]
            accL[s % 2] = partL
            rLs.start()

        rR_last, rL_last = rs_rdma(N_DEV - 2)
        rR_last.wait_recv()
        out_ref[:half, :] += rsR[N_DEV - 2]
        rL_last.wait_recv()
        out_ref[half:, :] += rsL[N_DEV - 2]

        for h in range(N_DEV - 1):
            aR, aL = ag_rdma(h)
            aR.wait_send()
            aL.wait_send()
        for s in (N_DEV - 3, N_DEV - 2):
            rR, rL = rs_rdma(s)
            rR.wait_send()
            rL.wait_send()

    return pl.pallas_call(
        body,
        out_shape=jax.ShapeDtypeStruct((m_per, d_model), jnp.float32),
        in_specs=[
            pl.BlockSpec(memory_space=pltpu.VMEM),
            pl.BlockSpec(memory_space=pltpu.VMEM),
            pl.BlockSpec(memory_space=pltpu.VMEM),
        ],
        out_specs=pl.BlockSpec(memory_space=pltpu.VMEM),
        scratch_shapes=[
            pltpu.VMEM((N_DEV, half, d_model), jnp.float32),
            pltpu.VMEM((N_DEV, half, d_model), jnp.float32),
            pltpu.VMEM((N_DEV - 1, half, d_model), jnp.float32),
            pltpu.VMEM((N_DEV - 1, half, d_model), jnp.float32),
            pltpu.VMEM((2, half, d_model), jnp.float32),
            pltpu.VMEM((2, half, d_model), jnp.float32),
            pltpu.SemaphoreType.DMA((N_DEV - 1,)),
            pltpu.SemaphoreType.DMA((N_DEV - 1,)),
            pltpu.SemaphoreType.DMA((N_DEV - 1,)),
            pltpu.SemaphoreType.DMA((N_DEV - 1,)),
            pltpu.SemaphoreType.DMA((N_DEV - 1,)),
            pltpu.SemaphoreType.DMA((N_DEV - 1,)),
            pltpu.SemaphoreType.DMA((N_DEV - 1,)),
            pltpu.SemaphoreType.DMA((N_DEV - 1,)),
        ],
        compiler_params=pltpu.CompilerParams(collective_id=0),
    )(x, W1, W2)


# device time: 94589 ns/iter; 2.2679x vs baseline; 1.0274x over previous
import jax
import jax.numpy as jnp
from jax import lax
from jax.experimental import pallas as pl
from jax.experimental.pallas import tpu as pltpu

N_DEV = 8


def kernel(x, W1, W2):
    m_per, d_model = x.shape
    half = m_per // 2
    quart = half // 2

    def body(x_ref, W1_ref, W2_ref, out_ref, xgR, xgL, rsR, rsL, accR, accL,
             agR_ssem, agR_rsem, agL_ssem, agL_rsem,
             rsR_ssem, rsR_rsem, rsL_ssem, rsL_rsem):
        my = lax.axis_index("i")
        left = lax.rem(my - 1 + N_DEV, N_DEV)
        right = lax.rem(my + 1, N_DEV)

        barrier_sem = pltpu.get_barrier_semaphore()
        for nbr in (left, right):
            pl.semaphore_signal(
                barrier_sem, inc=1,
                device_id=(nbr,), device_id_type=pl.DeviceIdType.MESH,
            )
        pl.semaphore_wait(barrier_sem, 2)

        def contribution(xblk):
            h = jnp.dot(xblk, W1_ref[...], preferred_element_type=jnp.float32)
            h = h * jax.nn.sigmoid(h)
            return jnp.dot(h, W2_ref[...], preferred_element_type=jnp.float32)

        def ag_rdma(h):
            blkR = (my - h) % N_DEV
            aR = pltpu.make_async_remote_copy(
                src_ref=xgR.at[blkR], dst_ref=xgR.at[blkR],
                send_sem=agR_ssem.at[h], recv_sem=agR_rsem.at[h],
                device_id=(right,), device_id_type=pl.DeviceIdType.MESH,
            )
            blkL = (my + h) % N_DEV
            aL = pltpu.make_async_remote_copy(
                src_ref=xgL.at[blkL], dst_ref=xgL.at[blkL],
                send_sem=agL_ssem.at[h], recv_sem=agL_rsem.at[h],
                device_id=(left,), device_id_type=pl.DeviceIdType.MESH,
            )
            return aR, aL

        def rs_rdma(s, sub):
            rR = pltpu.make_async_remote_copy(
                src_ref=accR.at[s % 2, sub], dst_ref=rsR.at[s, sub],
                send_sem=rsR_ssem.at[s, sub], recv_sem=rsR_rsem.at[s, sub],
                device_id=(right,), device_id_type=pl.DeviceIdType.MESH,
            )
            rL = pltpu.make_async_remote_copy(
                src_ref=accL.at[s % 2, sub], dst_ref=rsL.at[s, sub],
                send_sem=rsL_ssem.at[s, sub], recv_sem=rsL_rsem.at[s, sub],
                device_id=(left,), device_id_type=pl.DeviceIdType.MESH,
            )
            return rR, rL

        xgR[my] = x_ref[:half, :]
        xgL[my] = x_ref[half:, :]
        aR0, aL0 = ag_rdma(0)
        aR0.start()
        aL0.start()

        for s in range(N_DEV - 1):
            aRs, aLs = ag_rdma(s)
            aRs.wait_recv()
            aLs.wait_recv()
            if s < N_DEV - 2:
                aRn, aLn = ag_rdma(s + 1)
                aRn.start()
                aLn.start()

            partR = contribution(xgR[(my - 1 - s) % N_DEV])
            partL = contribution(xgL[(my + 1 + s) % N_DEV])
            if s == 0:
                out_ref[:half, :] = contribution(x_ref[:half, :])
                out_ref[half:, :] = contribution(x_ref[half:, :])

            if s > 1:
                for sub in (0, 1):
                    rR_old, rL_old = rs_rdma(s - 2, sub)
                    rR_old.wait_send()
                    rL_old.wait_send()

            for sub, lo, hi in ((0, 0, quart), (1, quart, half)):
                if s > 0:
                    rR_prev, rL_prev = rs_rdma(s - 1, sub)
                    rR_prev.wait_recv()
                    accR[s % 2, sub] = partR[lo:hi, :] + rsR[s - 1, sub]
                else:
                    accR[s % 2, sub] = partR[lo:hi, :]
                rRs, _ = rs_rdma(s, sub)
                rRs.start()
                if s > 0:
                    rL_prev.wait_recv()
                    accL[s % 2, sub] = partL[lo:hi, :] + rsL[s - 1, sub]
                else:
                    accL[s % 2, sub] = partL[lo:hi, :]
                _, rLs = rs_rdma(s, sub)
                rLs.start()

        for sub, lo, hi in ((0, 0, quart), (1, quart, half)):
            rR_last, rL_last = rs_rdma(N_DEV - 2, sub)
            rR_last.wait_recv()
            out_ref[lo:hi, :] += rsR[N_DEV - 2, sub]
            rL_last.wait_recv()
            out_ref[half + lo:half + hi, :] += rsL[N_DEV - 2, sub]

        for h in range(N_DEV - 1):
            aR, aL = ag_rdma(h)
            aR.wait_send()
            aL.wait_send()
        for s in (N_DEV - 3, N_DEV - 2):
            for sub in (0, 1):
                rR, rL = rs_rdma(s, sub)
                rR.wait_send()
                rL.wait_send()

    return pl.pallas_call(
        body,
        out_shape=jax.ShapeDtypeStruct((m_per, d_model), jnp.float32),
        in_specs=[
            pl.BlockSpec(memory_space=pltpu.VMEM),
            pl.BlockSpec(memory_space=pltpu.VMEM),
            pl.BlockSpec(memory_space=pltpu.VMEM),
        ],
        out_specs=pl.BlockSpec(memory_space=pltpu.VMEM),
        scratch_shapes=[
            pltpu.VMEM((N_DEV, half, d_model), jnp.float32),
            pltpu.VMEM((N_DEV, half, d_model), jnp.float32),
            pltpu.VMEM((N_DEV - 1, 2, quart, d_model), jnp.float32),
            pltpu.VMEM((N_DEV - 1, 2, quart, d_model), jnp.float32),
            pltpu.VMEM((2, 2, quart, d_model), jnp.float32),
            pltpu.VMEM((2, 2, quart, d_model), jnp.float32),
            pltpu.SemaphoreType.DMA((N_DEV - 1,)),
            pltpu.SemaphoreType.DMA((N_DEV - 1,)),
            pltpu.SemaphoreType.DMA((N_DEV - 1,)),
            pltpu.SemaphoreType.DMA((N_DEV - 1,)),
            pltpu.SemaphoreType.DMA((N_DEV - 1, 2)),
            pltpu.SemaphoreType.DMA((N_DEV - 1, 2)),
            pltpu.SemaphoreType.DMA((N_DEV - 1, 2)),
            pltpu.SemaphoreType.DMA((N_DEV - 1, 2)),
        ],
        compiler_params=pltpu.CompilerParams(collective_id=0),
    )(x, W1, W2)


# device time: 92873 ns/iter; 2.3098x vs baseline; 1.0185x over previous
import jax
import jax.numpy as jnp
from jax import lax
from jax.experimental import pallas as pl
from jax.experimental.pallas import tpu as pltpu

N_DEV = 8


def kernel(x, W1, W2):
    m_per, d_model = x.shape
    half = m_per // 2
    quart = half // 2

    def body(x_ref, W1_ref, W2_ref, out_ref, xgR, xgL, rsR, rsL, accR, accL,
             agR_ssem, agR_rsem, agL_ssem, agL_rsem,
             rsR_ssem, rsR_rsem, rsL_ssem, rsL_rsem):
        my = lax.axis_index("i")
        left = lax.rem(my - 1 + N_DEV, N_DEV)
        right = lax.rem(my + 1, N_DEV)

        barrier_sem = pltpu.get_barrier_semaphore()
        for nbr in (left, right):
            pl.semaphore_signal(
                barrier_sem, inc=1,
                device_id=(nbr,), device_id_type=pl.DeviceIdType.MESH,
            )
        pl.semaphore_wait(barrier_sem, 2)

        def contribution(xblk):
            h = jnp.dot(xblk, W1_ref[...], preferred_element_type=jnp.float32)
            h = h * jax.nn.sigmoid(h)
            return jnp.dot(h, W2_ref[...], preferred_element_type=jnp.float32)

        streams = (
            ("R", 0), ("L", 0), ("R", 1), ("L", 1),
        )

        def refs(dirn):
            if dirn == "R":
                return xgR, rsR, accR, agR_ssem, agR_rsem, rsR_ssem, rsR_rsem, right
            return xgL, rsL, accL, agL_ssem, agL_rsem, rsL_ssem, rsL_rsem, left

        def ag_blk(dirn, h):
            return (my - h) % N_DEV if dirn == "R" else (my + h) % N_DEV

        def rs_chunk(dirn, s):
            return (my - 1 - s) % N_DEV if dirn == "R" else (my + 1 + s) % N_DEV

        def ag_rdma(dirn, sub, h):
            xg, _, _, ssem, rsem, _, _, peer = refs(dirn)
            blk = ag_blk(dirn, h)
            return pltpu.make_async_remote_copy(
                src_ref=xg.at[blk, sub], dst_ref=xg.at[blk, sub],
                send_sem=ssem.at[h, sub], recv_sem=rsem.at[h, sub],
                device_id=(peer,), device_id_type=pl.DeviceIdType.MESH,
            )

        def rs_rdma(dirn, sub, s):
            _, rs, acc, _, _, ssem, rsem, peer = refs(dirn)
            return pltpu.make_async_remote_copy(
                src_ref=acc.at[s % 2, sub], dst_ref=rs.at[s, sub],
                send_sem=ssem.at[s, sub], recv_sem=rsem.at[s, sub],
                device_id=(peer,), device_id_type=pl.DeviceIdType.MESH,
            )

        xgR[my, 0] = x_ref[:quart, :]
        xgR[my, 1] = x_ref[quart:half, :]
        xgL[my, 0] = x_ref[half:half + quart, :]
        xgL[my, 1] = x_ref[half + quart:, :]
        for dirn, sub in streams:
            ag_rdma(dirn, sub, 0).start()

        for s in range(N_DEV - 1):
            for dirn, sub in streams:
                xg, rs, acc, _, _, _, _, _ = refs(dirn)
                ag_rdma(dirn, sub, s).wait_recv()
                if s < N_DEV - 2:
                    ag_rdma(dirn, sub, s + 1).start()
                part = contribution(xg[rs_chunk(dirn, s), sub])
                if s > 1:
                    rs_rdma(dirn, sub, s - 2).wait_send()
                if s > 0:
                    rs_rdma(dirn, sub, s - 1).wait_recv()
                    part = part + rs[s - 1, sub]
                acc[s % 2, sub] = part
                rs_rdma(dirn, sub, s).start()
            if s == 0:
                out_ref[:half, :] = contribution(x_ref[:half, :])
                out_ref[half:, :] = contribution(x_ref[half:, :])

        row0 = {("R", 0): 0, ("R", 1): quart, ("L", 0): half,
                ("L", 1): half + quart}
        for dirn, sub in streams:
            _, rs, _, _, _, _, _, _ = refs(dirn)
            rs_rdma(dirn, sub, N_DEV - 2).wait_recv()
            lo = row0[(dirn, sub)]
            out_ref[lo:lo + quart, :] += rs[N_DEV - 2, sub]

        for dirn, sub in streams:
            for h in range(N_DEV - 1):
                ag_rdma(dirn, sub, h).wait_send()
            for s in (N_DEV - 3, N_DEV - 2):
                rs_rdma(dirn, sub, s).wait_send()

    return pl.pallas_call(
        body,
        out_shape=jax.ShapeDtypeStruct((m_per, d_model), jnp.float32),
        in_specs=[
            pl.BlockSpec(memory_space=pltpu.VMEM),
            pl.BlockSpec(memory_space=pltpu.VMEM),
            pl.BlockSpec(memory_space=pltpu.VMEM),
        ],
        out_specs=pl.BlockSpec(memory_space=pltpu.VMEM),
        scratch_shapes=[
            pltpu.VMEM((N_DEV, 2, quart, d_model), jnp.float32),
            pltpu.VMEM((N_DEV, 2, quart, d_model), jnp.float32),
            pltpu.VMEM((N_DEV - 1, 2, quart, d_model), jnp.float32),
            pltpu.VMEM((N_DEV - 1, 2, quart, d_model), jnp.float32),
            pltpu.VMEM((2, 2, quart, d_model), jnp.float32),
            pltpu.VMEM((2, 2, quart, d_model), jnp.float32),
            pltpu.SemaphoreType.DMA((N_DEV - 1, 2)),
            pltpu.SemaphoreType.DMA((N_DEV - 1, 2)),
            pltpu.SemaphoreType.DMA((N_DEV - 1, 2)),
            pltpu.SemaphoreType.DMA((N_DEV - 1, 2)),
            pltpu.SemaphoreType.DMA((N_DEV - 1, 2)),
            pltpu.SemaphoreType.DMA((N_DEV - 1, 2)),
            pltpu.SemaphoreType.DMA((N_DEV - 1, 2)),
            pltpu.SemaphoreType.DMA((N_DEV - 1, 2)),
        ],
        compiler_params=pltpu.CompilerParams(collective_id=0),
    )(x, W1, W2)


# device time: 59401 ns/iter; 3.6113x vs baseline; 1.5635x over previous
import jax
import jax.numpy as jnp
from jax import lax
from jax.experimental import pallas as pl
from jax.experimental.pallas import tpu as pltpu

N_DEV = 8


def kernel(x, W1, W2):
    m_per, d_model = x.shape
    half = m_per // 2
    quart = half // 2

    def body(x_ref, W1_ref, W2_ref, out_ref, xgR, xgL, rsR, rsL, accR, accL,
             agR_ssem, agR_rsem, agL_ssem, agL_rsem,
             rsR_ssem, rsR_rsem, rsL_ssem, rsL_rsem):
        my = lax.axis_index("i")
        left = lax.rem(my - 1 + N_DEV, N_DEV)
        right = lax.rem(my + 1, N_DEV)

        barrier_sem = pltpu.get_barrier_semaphore()
        for nbr in (left, right):
            pl.semaphore_signal(
                barrier_sem, inc=1,
                device_id=(nbr,), device_id_type=pl.DeviceIdType.MESH,
            )
        pl.semaphore_wait(barrier_sem, 2)

        def contribution(xblk):
            h = jnp.dot(xblk, W1_ref[...], preferred_element_type=jnp.float32)
            h = h * jax.nn.sigmoid(h)
            return jnp.dot(h, W2_ref[...], preferred_element_type=jnp.float32)

        streams = (
            ("R", 0), ("L", 0), ("R", 1), ("L", 1),
        )

        def refs(dirn):
            if dirn == "R":
                return xgR, rsR, accR, agR_ssem, agR_rsem, rsR_ssem, rsR_rsem, right
            return xgL, rsL, accL, agL_ssem, agL_rsem, rsL_ssem, rsL_rsem, left

        def ag_blk(dirn, h):
            return (my - h) % N_DEV if dirn == "R" else (my + h) % N_DEV

        def rs_chunk(dirn, s):
            return (my - 1 - s) % N_DEV if dirn == "R" else (my + 1 + s) % N_DEV

        def ag_rdma(dirn, sub, h):
            xg, _, _, ssem, rsem, _, _, peer = refs(dirn)
            blk = ag_blk(dirn, h)
            return pltpu.make_async_remote_copy(
                src_ref=xg.at[blk, sub], dst_ref=xg.at[blk, sub],
                send_sem=ssem.at[h, sub], recv_sem=rsem.at[h, sub],
                device_id=(peer,), device_id_type=pl.DeviceIdType.MESH,
            )

        def rs_rdma(dirn, sub, s):
            _, rs, acc, _, _, ssem, rsem, peer = refs(dirn)
            return pltpu.make_async_remote_copy(
                src_ref=acc.at[s % 2, sub], dst_ref=rs.at[s, sub],
                send_sem=ssem.at[s, sub], recv_sem=rsem.at[s, sub],
                device_id=(peer,), device_id_type=pl.DeviceIdType.MESH,
            )

        xgR[my, 0] = x_ref[:quart, :].astype(jnp.bfloat16)
        xgR[my, 1] = x_ref[quart:half, :].astype(jnp.bfloat16)
        xgL[my, 0] = x_ref[half:half + quart, :].astype(jnp.bfloat16)
        xgL[my, 1] = x_ref[half + quart:, :].astype(jnp.bfloat16)
        for dirn, sub in streams:
            ag_rdma(dirn, sub, 0).start()

        for s in range(N_DEV - 1):
            for dirn, sub in streams:
                xg, rs, acc, _, _, _, _, _ = refs(dirn)
                ag_rdma(dirn, sub, s).wait_recv()
                if s < N_DEV - 2:
                    ag_rdma(dirn, sub, s + 1).start()
                part = contribution(xg[rs_chunk(dirn, s), sub])
                if s > 1:
                    rs_rdma(dirn, sub, s - 2).wait_send()
                if s > 0:
                    rs_rdma(dirn, sub, s - 1).wait_recv()
                    part = part + rs[s - 1, sub].astype(jnp.float32)
                acc[s % 2, sub] = part.astype(jnp.bfloat16)
                rs_rdma(dirn, sub, s).start()
            if s == 0:
                out_ref[:half, :] = contribution(x_ref[:half, :])
                out_ref[half:, :] = contribution(x_ref[half:, :])

        row0 = {("R", 0): 0, ("R", 1): quart, ("L", 0): half,
                ("L", 1): half + quart}
        for dirn, sub in streams:
            _, rs, _, _, _, _, _, _ = refs(dirn)
            rs_rdma(dirn, sub, N_DEV - 2).wait_recv()
            lo = row0[(dirn, sub)]
            out_ref[lo:lo + quart, :] += rs[N_DEV - 2, sub].astype(jnp.float32)

        for dirn, sub in streams:
            for h in range(N_DEV - 1):
                ag_rdma(dirn, sub, h).wait_send()
            for s in (N_DEV - 3, N_DEV - 2):
                rs_rdma(dirn, sub, s).wait_send()

    return pl.pallas_call(
        body,
        out_shape=jax.ShapeDtypeStruct((m_per, d_model), jnp.float32),
        in_specs=[
            pl.BlockSpec(memory_space=pltpu.VMEM),
            pl.BlockSpec(memory_space=pltpu.VMEM),
            pl.BlockSpec(memory_space=pltpu.VMEM),
        ],
        out_specs=pl.BlockSpec(memory_space=pltpu.VMEM),
        scratch_shapes=[
            pltpu.VMEM((N_DEV, 2, quart, d_model), jnp.bfloat16),
            pltpu.VMEM((N_DEV, 2, quart, d_model), jnp.bfloat16),
            pltpu.VMEM((N_DEV - 1, 2, quart, d_model), jnp.bfloat16),
            pltpu.VMEM((N_DEV - 1, 2, quart, d_model), jnp.bfloat16),
            pltpu.VMEM((2, 2, quart, d_model), jnp.bfloat16),
            pltpu.VMEM((2, 2, quart, d_model), jnp.bfloat16),
            pltpu.SemaphoreType.DMA((N_DEV - 1, 2)),
            pltpu.SemaphoreType.DMA((N_DEV - 1, 2)),
            pltpu.SemaphoreType.DMA((N_DEV - 1, 2)),
            pltpu.SemaphoreType.DMA((N_DEV - 1, 2)),
            pltpu.SemaphoreType.DMA((N_DEV - 1, 2)),
            pltpu.SemaphoreType.DMA((N_DEV - 1, 2)),
            pltpu.SemaphoreType.DMA((N_DEV - 1, 2)),
            pltpu.SemaphoreType.DMA((N_DEV - 1, 2)),
        ],
        compiler_params=pltpu.CompilerParams(collective_id=0),
    )(x, W1, W2)


# device time: 55006 ns/iter; 3.8998x vs baseline; 1.0799x over previous
import jax
import jax.numpy as jnp
from jax import lax
from jax.experimental import pallas as pl
from jax.experimental.pallas import tpu as pltpu

N_DEV = 8


def kernel(x, W1, W2):
    m_per, d_model = x.shape
    half = m_per // 2
    quart = half // 2

    def body(x_ref, W1_ref, W2_ref, out_ref, xgR, xgL, rsR, rsL, accR, accL,
             agR_ssem, agR_rsem, agL_ssem, agL_rsem,
             rsR_ssem, rsR_rsem, rsL_ssem, rsL_rsem):
        my = lax.axis_index("i")
        left = lax.rem(my - 1 + N_DEV, N_DEV)
        right = lax.rem(my + 1, N_DEV)

        barrier_sem = pltpu.get_barrier_semaphore()
        for nbr in (left, right):
            pl.semaphore_signal(
                barrier_sem, inc=1,
                device_id=(nbr,), device_id_type=pl.DeviceIdType.MESH,
            )
        pl.semaphore_wait(barrier_sem, 2)

        def contribution(xblk):
            h = jnp.dot(xblk, W1_ref[...], preferred_element_type=jnp.float32)
            h = h * jax.nn.sigmoid(h)
            return jnp.dot(h.astype(jnp.bfloat16), W2_ref[...],
                           preferred_element_type=jnp.float32)

        streams = (
            ("R", 0), ("L", 0), ("R", 1), ("L", 1),
        )

        def refs(dirn):
            if dirn == "R":
                return xgR, rsR, accR, agR_ssem, agR_rsem, rsR_ssem, rsR_rsem, right
            return xgL, rsL, accL, agL_ssem, agL_rsem, rsL_ssem, rsL_rsem, left

        def ag_blk(dirn, h):
            return (my - h) % N_DEV if dirn == "R" else (my + h) % N_DEV

        def rs_chunk(dirn, s):
            return (my - 1 - s) % N_DEV if dirn == "R" else (my + 1 + s) % N_DEV

        def ag_rdma(dirn, sub, h):
            xg, _, _, ssem, rsem, _, _, peer = refs(dirn)
            blk = ag_blk(dirn, h)
            return pltpu.make_async_remote_copy(
                src_ref=xg.at[blk, sub], dst_ref=xg.at[blk, sub],
                send_sem=ssem.at[h, sub], recv_sem=rsem.at[h, sub],
                device_id=(peer,), device_id_type=pl.DeviceIdType.MESH,
            )

        def rs_rdma(dirn, sub, s):
            _, rs, acc, _, _, ssem, rsem, peer = refs(dirn)
            return pltpu.make_async_remote_copy(
                src_ref=acc.at[s % 2, sub], dst_ref=rs.at[s, sub],
                send_sem=ssem.at[s, sub], recv_sem=rsem.at[s, sub],
                device_id=(peer,), device_id_type=pl.DeviceIdType.MESH,
            )

        xgR[my, 0] = x_ref[:quart, :].astype(jnp.bfloat16)
        xgR[my, 1] = x_ref[quart:half, :].astype(jnp.bfloat16)
        xgL[my, 0] = x_ref[half:half + quart, :].astype(jnp.bfloat16)
        xgL[my, 1] = x_ref[half + quart:, :].astype(jnp.bfloat16)
        for dirn, sub in streams:
            ag_rdma(dirn, sub, 0).start()

        for s in range(N_DEV - 1):
            for dirn, sub in streams:
                xg, rs, acc, _, _, _, _, _ = refs(dirn)
                ag_rdma(dirn, sub, s).wait_recv()
                if s < N_DEV - 2:
                    ag_rdma(dirn, sub, s + 1).start()
                part = contribution(xg[rs_chunk(dirn, s), sub])
                if s > 1:
                    rs_rdma(dirn, sub, s - 2).wait_send()
                if s > 0:
                    rs_rdma(dirn, sub, s - 1).wait_recv()
                    part = part + rs[s - 1, sub].astype(jnp.float32)
                acc[s % 2, sub] = part.astype(jnp.bfloat16)
                rs_rdma(dirn, sub, s).start()
            if s == 0:
                out_ref[:half, :] = contribution(x_ref[:half, :])
                out_ref[half:, :] = contribution(x_ref[half:, :])

        row0 = {("R", 0): 0, ("R", 1): quart, ("L", 0): half,
                ("L", 1): half + quart}
        for dirn, sub in streams:
            _, rs, _, _, _, _, _, _ = refs(dirn)
            rs_rdma(dirn, sub, N_DEV - 2).wait_recv()
            lo = row0[(dirn, sub)]
            out_ref[lo:lo + quart, :] += rs[N_DEV - 2, sub].astype(jnp.float32)

        for dirn, sub in streams:
            for h in range(N_DEV - 1):
                ag_rdma(dirn, sub, h).wait_send()
            for s in (N_DEV - 3, N_DEV - 2):
                rs_rdma(dirn, sub, s).wait_send()

    return pl.pallas_call(
        body,
        out_shape=jax.ShapeDtypeStruct((m_per, d_model), jnp.float32),
        in_specs=[
            pl.BlockSpec(memory_space=pltpu.VMEM),
            pl.BlockSpec(memory_space=pltpu.VMEM),
            pl.BlockSpec(memory_space=pltpu.VMEM),
        ],
        out_specs=pl.BlockSpec(memory_space=pltpu.VMEM),
        scratch_shapes=[
            pltpu.VMEM((N_DEV, 2, quart, d_model), jnp.bfloat16),
            pltpu.VMEM((N_DEV, 2, quart, d_model), jnp.bfloat16),
            pltpu.VMEM((N_DEV - 1, 2, quart, d_model), jnp.bfloat16),
            pltpu.VMEM((N_DEV - 1, 2, quart, d_model), jnp.bfloat16),
            pltpu.VMEM((2, 2, quart, d_model), jnp.bfloat16),
            pltpu.VMEM((2, 2, quart, d_model), jnp.bfloat16),
            pltpu.SemaphoreType.DMA((N_DEV - 1, 2)),
            pltpu.SemaphoreType.DMA((N_DEV - 1, 2)),
            pltpu.SemaphoreType.DMA((N_DEV - 1, 2)),
            pltpu.SemaphoreType.DMA((N_DEV - 1, 2)),
            pltpu.SemaphoreType.DMA((N_DEV - 1, 2)),
            pltpu.SemaphoreType.DMA((N_DEV - 1, 2)),
            pltpu.SemaphoreType.DMA((N_DEV - 1, 2)),
            pltpu.SemaphoreType.DMA((N_DEV - 1, 2)),
        ],
        compiler_params=pltpu.CompilerParams(collective_id=0),
    )(x.astype(jnp.bfloat16), W1.astype(jnp.bfloat16),
      W2.astype(jnp.bfloat16))
